# baseline (speedup 1.0000x reference)
import functools

import jax
import jax.numpy as jnp
from jax import lax
from jax.experimental import pallas as pl
from jax.experimental.pallas import tpu as pltpu
from jax.experimental.pallas import tpu_sc as plsc

BATCH = 16384
DIM = 2048


TC_ROWS = 2048
LANES = 128
NBLK = DIM // LANES
TC_BLOCK_ROWS = 64


def _tc_body(idx_ref, z_ref, o_ref):
    idx = idx_ref[0, 0, :]
    local = jnp.bitwise_and(idx, LANES - 1)
    srcb = jnp.right_shift(idx, 7)
    for rs in range(TC_BLOCK_ROWS // 8):
        zsub = z_ref[rs * 8:(rs + 1) * 8, :]
        for ob in range(NBLK):
            loc2 = jnp.broadcast_to(
                local[None, ob * LANES:(ob + 1) * LANES], (8, LANES))
            sb2 = jnp.broadcast_to(
                srcb[None, ob * LANES:(ob + 1) * LANES], (8, LANES))
            level = [
                jnp.take_along_axis(
                    zsub[:, ib * LANES:(ib + 1) * LANES], loc2, axis=1)
                for ib in range(NBLK)
            ]
            for k in range(4):
                bit = jnp.bitwise_and(jnp.right_shift(sb2, k), 1) == 1
                level = [
                    jnp.where(bit, level[2 * i + 1], level[2 * i])
                    for i in range(len(level) // 2)
                ]
            o_ref[rs * 8:(rs + 1) * 8, ob * LANES:(ob + 1) * LANES] = level[0]


def _tc_kernel(z, perm_indices):
    idx3 = perm_indices.reshape(1, 1, DIM)
    return pl.pallas_call(
        _tc_body,
        grid=(TC_ROWS // TC_BLOCK_ROWS,),
        in_specs=[
            pl.BlockSpec((1, 1, DIM), lambda i: (0, 0, 0)),
            pl.BlockSpec((TC_BLOCK_ROWS, DIM), lambda i: (i, 0)),
        ],
        out_specs=pl.BlockSpec((TC_BLOCK_ROWS, DIM), lambda i: (i, 0)),
        out_shape=jax.ShapeDtypeStruct((TC_ROWS, DIM), z.dtype),
    )(idx3, z)


L = 16
NC = 2
NS = 16
NW = NC * NS
SC_ROWS = BATCH - TC_ROWS
ROWS_PER_W = SC_ROWS // NW
CHUNK_R = 8
NCHUNK = ROWS_PER_W // CHUNK_R
CHUNK_BYTES = CHUNK_R * DIM * 4
assert NCHUNK % 2 == 0


def _sc_body(z_hbm, idx_hbm, out_hbm, idx_v, in_bufs, out_bufs, sems_i, sems_o):
    wid = lax.axis_index("s") * NC + lax.axis_index("c")
    row0 = TC_ROWS + wid * ROWS_PER_W
    pltpu.sync_copy(idx_hbm, idx_v)

    def in_slice(c):
        return z_hbm.at[pl.ds(row0 + c * CHUNK_R, CHUNK_R)]

    def out_slice(c):
        return out_hbm.at[pl.ds(row0 + c * CHUNK_R, CHUNK_R)]

    def compute(in_v, out_v):
        @plsc.parallel_loop(0, DIM // L, unroll=4)
        def col_body(k):
            colv = idx_v[pl.ds(k * L, L)]
            for r in range(CHUNK_R):
                rsplat = jnp.full((L,), r, jnp.int32)
                vals = plsc.load_gather(in_v, [rsplat, colv])
                out_v[r, pl.ds(k * L, L)] = vals

    for b in range(2):
        pltpu.async_copy(in_slice(b), in_bufs[b], sems_i[b])

    def pair_body(p, carry):
        for b in range(2):
            c = 2 * p + b
            pltpu.make_async_copy(in_slice(0), in_bufs[b], sems_i[b]).wait()

            @pl.when(p > 0)
            def _wait_out():
                pltpu.make_async_copy(out_bufs[b], out_slice(0), sems_o[b]).wait()

            compute(in_bufs[b], out_bufs[b])
            pltpu.async_copy(out_bufs[b], out_slice(c), sems_o[b])

            @pl.when(c + 2 < NCHUNK)
            def _prefetch():
                pltpu.async_copy(in_slice(c + 2), in_bufs[b], sems_i[b])

        return carry

    lax.fori_loop(0, NCHUNK // 2, pair_body, 0)

    for b in range(2):
        pltpu.make_async_copy(out_bufs[b], out_slice(0), sems_o[b]).wait()


_sc_kernel = functools.partial(
    pl.kernel,
    mesh=plsc.VectorSubcoreMesh(core_axis_name="c", subcore_axis_name="s"),
    out_type=jax.ShapeDtypeStruct((BATCH, DIM), jnp.float32),
    compiler_params=pltpu.CompilerParams(needs_layout_passes=False),
    scratch_types=[
        pltpu.VMEM((DIM,), jnp.int32),
        [pltpu.VMEM((CHUNK_R, DIM), jnp.float32) for _ in range(2)],
        [pltpu.VMEM((CHUNK_R, DIM), jnp.float32) for _ in range(2)],
        [pltpu.SemaphoreType.DMA for _ in range(2)],
        [pltpu.SemaphoreType.DMA for _ in range(2)],
    ],
)(_sc_body)


def kernel(z, perm_indices):
    sc_out = _sc_kernel(z, perm_indices)
    tc_out = _tc_kernel(z, perm_indices)
    return lax.dynamic_update_slice(sc_out, tc_out, (0, 0))


# speedup vs baseline: 1.1024x; 1.1024x over previous
import functools

import jax
import jax.numpy as jnp
from jax import lax
from jax.experimental import pallas as pl
from jax.experimental.pallas import tpu as pltpu
from jax.experimental.pallas import tpu_sc as plsc

BATCH = 16384
DIM = 2048
L = 16
NC = 2
NS = 16
NW = NC * NS
ROWS_PER_W = BATCH // NW
CHUNK_R = 4
NCHUNK = ROWS_PER_W // CHUNK_R
NBUF = 4
assert NCHUNK % NBUF == 0


def _sc_body(z_hbm, idx_hbm, out_hbm, idx_v, in_bufs, out_bufs, sems_i, sems_o):
    wid = lax.axis_index("s") * NC + lax.axis_index("c")
    row0 = wid * ROWS_PER_W
    pltpu.sync_copy(idx_hbm, idx_v)

    def in_slice(c):
        return z_hbm.at[pl.ds(row0 + c * CHUNK_R, CHUNK_R)]

    def out_slice(c):
        return out_hbm.at[pl.ds(row0 + c * CHUNK_R, CHUNK_R)]

    def compute(in_v, out_v):
        @plsc.parallel_loop(0, DIM // L, unroll=4)
        def col_body(k):
            colv = idx_v[pl.ds(k * L, L)]
            for r in range(CHUNK_R):
                rsplat = jnp.full((L,), r, jnp.int32)
                vals = plsc.load_gather(in_v, [rsplat, colv])
                out_v[r, pl.ds(k * L, L)] = vals

    for b in range(NBUF):
        pltpu.async_copy(in_slice(b), in_bufs[b], sems_i[b])

    def ring_body(p, carry):
        for b in range(NBUF):
            c = NBUF * p + b
            pltpu.make_async_copy(in_slice(0), in_bufs[b], sems_i[b]).wait()

            @pl.when(p > 0)
            def _wait_out():
                pltpu.make_async_copy(out_bufs[b], out_slice(0), sems_o[b]).wait()

            compute(in_bufs[b], out_bufs[b])
            pltpu.async_copy(out_bufs[b], out_slice(c), sems_o[b])

            @pl.when(c + NBUF < NCHUNK)
            def _prefetch():
                pltpu.async_copy(in_slice(c + NBUF), in_bufs[b], sems_i[b])

        return carry

    lax.fori_loop(0, NCHUNK // NBUF, ring_body, 0)

    for b in range(NBUF):
        pltpu.make_async_copy(out_bufs[b], out_slice(0), sems_o[b]).wait()


_sc_kernel = functools.partial(
    pl.kernel,
    mesh=plsc.VectorSubcoreMesh(core_axis_name="c", subcore_axis_name="s"),
    out_type=jax.ShapeDtypeStruct((BATCH, DIM), jnp.float32),
    compiler_params=pltpu.CompilerParams(needs_layout_passes=False),
    scratch_types=[
        pltpu.VMEM((DIM,), jnp.int32),
        [pltpu.VMEM((CHUNK_R, DIM), jnp.float32) for _ in range(NBUF)],
        [pltpu.VMEM((CHUNK_R, DIM), jnp.float32) for _ in range(NBUF)],
        [pltpu.SemaphoreType.DMA for _ in range(NBUF)],
        [pltpu.SemaphoreType.DMA for _ in range(NBUF)],
    ],
)(_sc_body)


def kernel(z, perm_indices):
    return _sc_kernel(z, perm_indices)


# speedup vs baseline: 1.1126x; 1.0093x over previous
import functools

import jax
import jax.numpy as jnp
from jax import lax
from jax.experimental import pallas as pl
from jax.experimental.pallas import tpu as pltpu
from jax.experimental.pallas import tpu_sc as plsc

BATCH = 16384
DIM = 2048
L = 16
NC = 2
NS = 16
NW = NC * NS
ROWS_PER_W = BATCH // NW
CHUNK_R = 4
NCHUNK = ROWS_PER_W // CHUNK_R
NBUF = 4
assert NCHUNK % NBUF == 0


def _sc_body(z_hbm, idx_hbm, out_hbm, idx_v, in_bufs, out_bufs, sems_i, sems_o):
    wid = lax.axis_index("s") * NC + lax.axis_index("c")
    row0 = wid * ROWS_PER_W

    def in_slice(c):
        return z_hbm.at[pl.ds(row0 + c * CHUNK_R, CHUNK_R)]

    def out_slice(c):
        return out_hbm.at[pl.ds(row0 + c * CHUNK_R, CHUNK_R)]

    def compute(in_v, out_v):
        @plsc.parallel_loop(0, DIM // L, unroll=8)
        def col_body(k):
            colv = idx_v[pl.ds(k * L, L)]
            for r in range(CHUNK_R):
                rsplat = jnp.full((L,), r, jnp.int32)
                vals = plsc.load_gather(in_v, [rsplat, colv])
                out_v[r, pl.ds(k * L, L)] = vals

    for b in range(NBUF):
        pltpu.async_copy(in_slice(b), in_bufs[b], sems_i[b])
    pltpu.sync_copy(idx_hbm, idx_v)

    def ring_body(p, carry):
        for b in range(NBUF):
            c = NBUF * p + b
            pltpu.make_async_copy(in_slice(0), in_bufs[b], sems_i[b]).wait()

            @pl.when(p > 0)
            def _wait_out():
                pltpu.make_async_copy(out_bufs[b], out_slice(0), sems_o[b]).wait()

            compute(in_bufs[b], out_bufs[b])
            pltpu.async_copy(out_bufs[b], out_slice(c), sems_o[b])

            @pl.when(c + NBUF < NCHUNK)
            def _prefetch():
                pltpu.async_copy(in_slice(c + NBUF), in_bufs[b], sems_i[b])

        return carry

    lax.fori_loop(0, NCHUNK // NBUF, ring_body, 0)

    for b in range(NBUF):
        pltpu.make_async_copy(out_bufs[b], out_slice(0), sems_o[b]).wait()


_sc_kernel = functools.partial(
    pl.kernel,
    mesh=plsc.VectorSubcoreMesh(core_axis_name="c", subcore_axis_name="s"),
    out_type=jax.ShapeDtypeStruct((BATCH, DIM), jnp.float32),
    compiler_params=pltpu.CompilerParams(needs_layout_passes=False),
    scratch_types=[
        pltpu.VMEM((DIM,), jnp.int32),
        [pltpu.VMEM((CHUNK_R, DIM), jnp.float32) for _ in range(NBUF)],
        [pltpu.VMEM((CHUNK_R, DIM), jnp.float32) for _ in range(NBUF)],
        [pltpu.SemaphoreType.DMA for _ in range(NBUF)],
        [pltpu.SemaphoreType.DMA for _ in range(NBUF)],
    ],
)(_sc_body)


def kernel(z, perm_indices):
    return _sc_kernel(z, perm_indices)
